# baseline (speedup 1.0000x reference)
import jax
import jax.numpy as jnp
from jax.experimental import pallas as pl
from jax.experimental.pallas import tpu as pltpu


def _unpool_body(x_ref, o_ref):
    _, Hb, W, C = x_ref.shape
    rs = jax.lax.broadcasted_iota(jnp.int32, (2 * W, W), 0)
    cs = jax.lax.broadcasted_iota(jnp.int32, (2 * W, W), 1)
    P = (rs == 2 * cs).astype(jnp.float32)
    z = jnp.zeros((2 * W, C), x_ref.dtype)
    for h in range(Hb):
        xh = x_ref[0, h]
        row = jax.lax.dot_general(
            P, xh, (((1,), (0,)), ((), ())),
            precision=jax.lax.Precision.HIGHEST,
            preferred_element_type=jnp.float32)
        o_ref[0, 2 * h] = row
        o_ref[0, 2 * h + 1] = z


def kernel(inputs):
    B, H, W, C = inputs.shape
    Hb = 8
    grid = (B, H // Hb)
    return pl.pallas_call(
        _unpool_body,
        grid=grid,
        in_specs=[pl.BlockSpec((1, Hb, W, C), lambda b, i: (b, i, 0, 0))],
        out_specs=pl.BlockSpec((1, 2 * Hb, 2 * W, C),
                               lambda b, i: (b, i, 0, 0)),
        out_shape=jax.ShapeDtypeStruct((B, 2 * H, 2 * W, C), inputs.dtype),
        compiler_params=pltpu.CompilerParams(
            dimension_semantics=("parallel", "arbitrary")),
    )(inputs)


# speedup vs baseline: 1.0520x; 1.0520x over previous
import jax
import jax.numpy as jnp
from jax.experimental import pallas as pl
from jax.experimental.pallas import tpu as pltpu


def _unpool_body(x_ref, o_ref):
    _, Hb, W, C = x_ref.shape
    rs = jax.lax.broadcasted_iota(jnp.int32, (2 * W, W), 0)
    cs = jax.lax.broadcasted_iota(jnp.int32, (2 * W, W), 1)
    P = (rs == 2 * cs).astype(jnp.float32)
    z = jnp.zeros((2 * W, C), x_ref.dtype)
    for h in range(Hb):
        xh = x_ref[0, h]
        row = jax.lax.dot_general(
            P, xh, (((1,), (0,)), ((), ())),
            precision=jax.lax.Precision.HIGHEST,
            preferred_element_type=jnp.float32)
        o_ref[0, 2 * h] = row
        o_ref[0, 2 * h + 1] = z


def kernel(inputs):
    B, H, W, C = inputs.shape
    Hb = 16
    grid = (B, H // Hb)
    return pl.pallas_call(
        _unpool_body,
        grid=grid,
        in_specs=[pl.BlockSpec((1, Hb, W, C), lambda b, i: (b, i, 0, 0))],
        out_specs=pl.BlockSpec((1, 2 * Hb, 2 * W, C),
                               lambda b, i: (b, i, 0, 0)),
        out_shape=jax.ShapeDtypeStruct((B, 2 * H, 2 * W, C), inputs.dtype),
        compiler_params=pltpu.CompilerParams(
            dimension_semantics=("parallel", "arbitrary")),
    )(inputs)


# speedup vs baseline: 1.0644x; 1.0117x over previous
import jax
import jax.numpy as jnp
from jax.experimental import pallas as pl
from jax.experimental.pallas import tpu as pltpu


def _unpool_body(x_ref, o_ref):
    _, Hb, W, C = x_ref.shape
    rs = jax.lax.broadcasted_iota(jnp.int32, (2 * W, W), 0)
    cs = jax.lax.broadcasted_iota(jnp.int32, (2 * W, W), 1)
    P = (rs == 2 * cs).astype(jnp.float32)
    z = jnp.zeros((2 * W, C), x_ref.dtype)
    for h in range(Hb):
        xh = x_ref[0, h]
        row = jax.lax.dot_general(
            P, xh, (((1,), (0,)), ((), ())),
            precision=jax.lax.Precision.HIGHEST,
            preferred_element_type=jnp.float32)
        o_ref[0, 2 * h] = row
        o_ref[0, 2 * h + 1] = z


def kernel(inputs):
    B, H, W, C = inputs.shape
    Hb = 28
    grid = (B, H // Hb)
    return pl.pallas_call(
        _unpool_body,
        grid=grid,
        in_specs=[pl.BlockSpec((1, Hb, W, C), lambda b, i: (b, i, 0, 0))],
        out_specs=pl.BlockSpec((1, 2 * Hb, 2 * W, C),
                               lambda b, i: (b, i, 0, 0)),
        out_shape=jax.ShapeDtypeStruct((B, 2 * H, 2 * W, C), inputs.dtype),
        compiler_params=pltpu.CompilerParams(
            dimension_semantics=("parallel", "arbitrary")),
    )(inputs)
